# baseline (speedup 1.0000x reference)
import functools
import jax
import jax.numpy as jnp
from jax import lax
from jax.experimental import pallas as pl
from jax.experimental.pallas import tpu as pltpu
from jax.experimental.pallas import tpu_sc as plsc

NUM_CORES = 2
NUM_SUBCORES = 16
LANES = 16
NW = NUM_CORES * NUM_SUBCORES

BATCH = 16384
EMBED_DIM = 32
B_PER_W = BATCH // NW
CHUNK = 128
N_CHUNK = B_PER_W // CHUNK
N_BLOCKS = B_PER_W // LANES


def _gmf_body(uid_hbm, iid_hbm, ut_hbm, it_hbm, w_hbm, b_hbm, out_hbm,
              idxu_v, idxi_v, urows_v, irows_v, w_v, b_v, out_v, sem):
    wid = lax.axis_index("c") * NUM_SUBCORES + lax.axis_index("s")
    base = wid * B_PER_W
    crow = wid * N_CHUNK

    pltpu.sync_copy(uid_hbm.at[pl.ds(crow, N_CHUNK)], idxu_v)
    pltpu.sync_copy(iid_hbm.at[pl.ds(crow, N_CHUNK)], idxi_v)
    pltpu.sync_copy(w_hbm, w_v)
    pltpu.sync_copy(b_hbm, b_v)

    copies = []
    for j in range(N_CHUNK):
        copies.append(pltpu.async_copy(
            ut_hbm.at[idxu_v.at[j]], urows_v.at[pl.ds(j * CHUNK, CHUNK)], sem))
        copies.append(pltpu.async_copy(
            it_hbm.at[idxi_v.at[j]], irows_v.at[pl.ds(j * CHUNK, CHUNK)], sem))
    for c in copies:
        c.wait()

    w_lo = w_v[pl.ds(0, LANES)]
    w_hi = w_v[pl.ds(LANES, LANES)]
    bias = b_v[...]
    lane_iota = lax.broadcasted_iota(jnp.int32, (LANES,), 0)

    def block(blk, carry):
        rbase = blk * LANES
        for r in range(LANES):
            row = rbase + r
            irows_v[row, pl.ds(0, LANES)] = irows_v[row, pl.ds(0, LANES)] * w_lo
            irows_v[row, pl.ds(LANES, LANES)] = (
                irows_v[row, pl.ds(LANES, LANES)] * w_hi)
        b_idx = rbase + lane_iota
        acc = bias
        for d in range(EMBED_DIM):
            d_idx = jnp.full((LANES,), d, jnp.int32)
            ug = plsc.load_gather(urows_v, [b_idx, d_idx])
            ig = plsc.load_gather(irows_v, [b_idx, d_idx])
            acc = acc + ug * ig
        out_v[pl.ds(rbase, LANES)] = acc
        return carry

    lax.fori_loop(0, N_BLOCKS, block, 0)

    pltpu.sync_copy(out_v, out_hbm.at[pl.ds(base, B_PER_W)])


@jax.jit
def _gmf(user_ids, item_ids, user_table, item_table, fc_w32, fc_b16):
    mesh = plsc.VectorSubcoreMesh(
        core_axis_name="c", subcore_axis_name="s",
        num_cores=NUM_CORES, num_subcores=NUM_SUBCORES)
    f = pl.kernel(
        _gmf_body,
        out_type=jax.ShapeDtypeStruct((BATCH,), jnp.float32),
        mesh=mesh,
        compiler_params=pltpu.CompilerParams(
            needs_layout_passes=False, use_tc_tiling_on_sc=False),
        scratch_types=[
            pltpu.VMEM((N_CHUNK, CHUNK), jnp.int32),
            pltpu.VMEM((N_CHUNK, CHUNK), jnp.int32),
            pltpu.VMEM((B_PER_W, EMBED_DIM), jnp.float32),
            pltpu.VMEM((B_PER_W, EMBED_DIM), jnp.float32),
            pltpu.VMEM((EMBED_DIM,), jnp.float32),
            pltpu.VMEM((LANES,), jnp.float32),
            pltpu.VMEM((B_PER_W,), jnp.float32),
            pltpu.SemaphoreType.DMA,
        ],
    )
    return f(user_ids.reshape(BATCH // CHUNK, CHUNK),
             item_ids.reshape(BATCH // CHUNK, CHUNK),
             user_table, item_table, fc_w32, fc_b16)


def kernel(user_ids, item_ids, user_table, item_table, fc_w, fc_b):
    fc_w32 = fc_w.reshape(EMBED_DIM)
    fc_b16 = jnp.broadcast_to(fc_b, (LANES,))
    return _gmf(user_ids.astype(jnp.int32), item_ids.astype(jnp.int32),
                user_table, item_table, fc_w32, fc_b16)
